# speedup vs baseline: 1.3369x; 1.3369x over previous
import functools

import jax
import jax.numpy as jnp
from jax import lax
from jax.experimental import pallas as pl
from jax.experimental.pallas import tpu as pltpu
from jax.experimental.pallas import tpu_sc as plsc

_HIDDEN = 1024
_NC = 2
_NS = 16
_NW = _NC * _NS
_CHUNK = 32
_NCHUNK = 8
_BPW = _CHUNK * _NCHUNK
_B = _BPW * _NW

_mesh = plsc.VectorSubcoreMesh(core_axis_name="c", subcore_axis_name="s")


@functools.partial(
    pl.kernel,
    mesh=_mesh,
    out_type=jax.ShapeDtypeStruct((_B, _HIDDEN), jnp.float32),
    scratch_types=[
        pltpu.VMEM((_NCHUNK, _CHUNK), jnp.int32),
        pltpu.VMEM((_CHUNK, _HIDDEN), jnp.float32),
        pltpu.VMEM((_CHUNK, _HIDDEN), jnp.float32),
        pltpu.SemaphoreType.DMA,
        pltpu.SemaphoreType.DMA,
        pltpu.SemaphoreType.DMA,
        pltpu.SemaphoreType.DMA,
    ],
)
def _gather_rows(ids_hbm, table_hbm, out_hbm, idx_v, buf0, buf1, g0, g1, p0, p1):
    wid = lax.axis_index("s") * _NC + lax.axis_index("c")
    base = wid * _BPW
    pltpu.sync_copy(ids_hbm.at[wid], idx_v)
    bufs = (buf0, buf1)
    gsems = (g0, g1)
    psems = (p0, p1)
    gathers = [None, None]
    puts = [None, None]
    gathers[0] = pltpu.async_copy(table_hbm.at[idx_v.at[0]], buf0, g0)
    for j in range(_NCHUNK):
        b = j & 1
        gathers[b].wait()
        if j + 1 < _NCHUNK:
            nb = (j + 1) & 1
            if puts[nb] is not None:
                puts[nb].wait()
            gathers[nb] = pltpu.async_copy(
                table_hbm.at[idx_v.at[j + 1]], bufs[nb], gsems[nb])
        puts[b] = pltpu.async_copy(
            bufs[b], out_hbm.at[pl.ds(base + j * _CHUNK, _CHUNK)], psems[b])
    puts[0].wait()
    puts[1].wait()


def kernel(input_ids, position_ids, attention_mask, word_embeddings):
    batch, seq = input_ids.shape
    ids = input_ids.astype(jnp.int32).reshape(_NW, _NCHUNK, _CHUNK)
    rows = _gather_rows(ids, word_embeddings)
    hidden_states = rows.reshape(batch, seq, _HIDDEN)
    return hidden_states, position_ids, attention_mask.astype(bool)
